# Initial kernel scaffold; baseline (speedup 1.0000x reference)
import jax
import jax.numpy as jnp
from jax.experimental import pallas as pl


def kernel(x, pos, edge_index, edge_vec, batch, fcw1, fcb1, fcw2x, fcw2s, Wsh, Wself, Wout, gamma, beta):
    raise NotImplementedError("write your pallas kernel here")


# speedup vs baseline: 2.0575x; 2.0575x over previous
import functools

import jax
import jax.numpy as jnp
from jax import lax
from jax.experimental import pallas as pl
from jax.experimental.pallas import tpu as pltpu
from jax.experimental.pallas import tpu_sc as plsc

_N = 10000
_E = 320000
_D = 128
_NB = 10
_MAXR = 3.5
_NNEI = 32.0
_HID = 100

_NCORE = 2
_NSUB = 16
_NW = _NCORE * _NSUB
_EPW = _E // _NW
_CHI = 80
_IDXROWS = _E // _CHI
_RPW = _IDXROWS // _NW
_GRP = 5
_NGRP = _RPW // _GRP
_GE = _GRP * _CHI
_NPAD = 10240
_NPT = _NPAD // _NSUB

@functools.cache
def _sc_gather_fn():
    mesh = plsc.VectorSubcoreMesh(core_axis_name="c", subcore_axis_name="s",
                                  num_cores=_NCORE, num_subcores=_NSUB)
    return pl.kernel(
        _sc_gather_body,
        out_type=jax.ShapeDtypeStruct((_E, _D), jnp.float32),
        mesh=mesh,
        scratch_types=[
            pltpu.VMEM((_RPW, _CHI), jnp.int32),
            pltpu.VMEM((_GE, _D), jnp.float32),
            pltpu.SemaphoreType.DMA,
        ],
    )


def _sc_gather(x, src):
    return _sc_gather_fn()(x, src)


def _sc_gather_body(x_hbm, src_hbm, xg_hbm, idxv, buf, sem):
    cid = lax.axis_index("c")
    sid = lax.axis_index("s")
    wid = cid * _NSUB + sid
    pltpu.sync_copy(src_hbm.at[wid], idxv)

    def grp(g, carry):
        descs = []
        for k in range(_GRP):
            descs.append(pltpu.async_copy(
                x_hbm.at[idxv.at[g * _GRP + k]],
                buf.at[pl.ds(k * _CHI, _CHI)], sem))
        for d in descs:
            d.wait()
        pltpu.sync_copy(buf, xg_hbm.at[pl.ds(wid * _EPW + g * _GE, _GE)])
        return carry

    lax.fori_loop(0, _NGRP, grp, 0)


@functools.cache
def _sc_scatter_fn():
    mesh = plsc.VectorSubcoreMesh(core_axis_name="c", subcore_axis_name="s",
                                  num_cores=_NCORE, num_subcores=_NSUB)
    return pl.kernel(
        _sc_scatter_body,
        out_type=jax.ShapeDtypeStruct((_NCORE * _NPAD, _D), jnp.float32),
        mesh=mesh,
        scratch_types=[
            pltpu.VMEM((_RPW, _CHI), jnp.int32),
            pltpu.VMEM((_CHI, _D), jnp.float32),
            pltpu.VMEM_SHARED((_NPAD, _D), jnp.float32),
            pltpu.SemaphoreType.DMA,
        ],
    )


def _sc_scatter(m, dst, zer):
    return _sc_scatter_fn()(m, dst, zer)


def _sc_scatter_body(m_hbm, dst_hbm, zer_hbm, out_hbm, idxv, buf, aggsh, sem):
    cid = lax.axis_index("c")
    sid = lax.axis_index("s")
    wid = cid * _NSUB + sid
    pltpu.sync_copy(zer_hbm, aggsh.at[pl.ds(sid * _NPT, _NPT)])
    pltpu.sync_copy(dst_hbm.at[wid], idxv)
    plsc.subcore_barrier()

    def grp(g, carry):
        pltpu.sync_copy(m_hbm.at[pl.ds(wid * _EPW + g * _CHI, _CHI)], buf)
        pltpu.sync_copy(buf, aggsh.at[idxv.at[g]], add=True)
        return carry

    lax.fori_loop(0, _RPW, grp, 0)
    plsc.subcore_barrier()
    for t in range(8):
        off = sid * _NPT + t * _CHI
        pltpu.sync_copy(aggsh.at[pl.ds(off, _CHI)], buf)
        pltpu.sync_copy(buf, out_hbm.at[pl.ds(cid * _NPAD + off, _CHI)])


_BE = 6400


def _edge_body(ev_ref, xg_ref, fcw1_ref, fcb1_ref, fcw2x_ref, fcw2s_ref,
               wsh_ref, m_ref):
    ev = ev_ref[...]
    l2 = jnp.sum(ev * ev, axis=1, keepdims=True)
    ln = jnp.sqrt(l2)
    unit = ev / (ln + 1e-12)
    ea = jnp.concatenate(
        [jnp.ones((_BE, 1), jnp.float32), jnp.sqrt(3.0) * unit], axis=1)
    step = _MAXR / (_NB + 1)
    k = lax.broadcasted_iota(jnp.int32, (_BE, _NB), 1).astype(jnp.float32)
    diff = (ln - (k + 1.0) * step) / step
    mask = ((diff > -1.0) & (diff < 1.0)).astype(jnp.float32)
    emb = jnp.cos(jnp.pi / 2.0 * diff) * mask * (_NB ** 0.5)
    h = jnp.maximum(
        jnp.dot(emb, fcw1_ref[...], preferred_element_type=jnp.float32)
        + fcb1_ref[...], 0.0)
    a = jnp.dot(h, fcw2x_ref[...], preferred_element_type=jnp.float32)
    c = (jnp.dot(ea, wsh_ref[...], preferred_element_type=jnp.float32)
         * jnp.dot(h, fcw2s_ref[...], preferred_element_type=jnp.float32))
    m_ref[...] = xg_ref[...] * a + c


def _edge_messages(edge_vec, xg, fcw1, fcb1, fcw2x, fcw2s, wsh):
    grid = _E // _BE
    full = lambda shape: pl.BlockSpec(shape, lambda i: (0, 0))
    return pl.pallas_call(
        _edge_body,
        grid=(grid,),
        in_specs=[
            pl.BlockSpec((_BE, 3), lambda i: (i, 0)),
            pl.BlockSpec((_BE, _D), lambda i: (i, 0)),
            full((_NB, _HID)),
            full((1, _HID)),
            full((_HID, _D)),
            full((_HID, _D)),
            full((4, _D)),
        ],
        out_specs=pl.BlockSpec((_BE, _D), lambda i: (i, 0)),
        out_shape=jax.ShapeDtypeStruct((_E, _D), jnp.float32),
    )(edge_vec, xg, fcw1, fcb1, fcw2x, fcw2s, wsh)


def _finish_body(x_ref, p_ref, wself_ref, wout_ref, gamma_ref, beta_ref,
                 o_ref):
    agg = (p_ref[0] + p_ref[1]) * (1.0 / jnp.sqrt(_NNEI))
    out = (jnp.dot(x_ref[...], wself_ref[...],
                   preferred_element_type=jnp.float32)
           + jnp.dot(agg, wout_ref[...], preferred_element_type=jnp.float32))
    mean = jnp.mean(out, axis=0, keepdims=True)
    cent = out - mean
    var = jnp.mean(cent * cent, axis=0, keepdims=True)
    o_ref[...] = gamma_ref[...] * cent * lax.rsqrt(var + 1e-5) + beta_ref[...]


def _finish(x, partials, wself, wout, gamma, beta):
    return pl.pallas_call(
        _finish_body,
        out_shape=jax.ShapeDtypeStruct((_N, _D), jnp.float32),
    )(x, partials, wself, wout, gamma, beta)


def kernel(x, pos, edge_index, edge_vec, batch, fcw1, fcb1, fcw2x, fcw2s,
           Wsh, Wself, Wout, gamma, beta):
    src = edge_index[0].reshape(_NW, _RPW, _CHI)
    dst = edge_index[1].reshape(_NW, _RPW, _CHI)
    xg = _sc_gather(x, src)
    m = _edge_messages(edge_vec, xg, fcw1, fcb1.reshape(1, _HID),
                       fcw2x, fcw2s, Wsh)
    zer = jnp.zeros((_NPT, _D), jnp.float32)
    partials = _sc_scatter(m, dst, zer)
    out = _finish(x, partials.reshape(_NCORE, _NPAD, _D)[:, :_N], Wself, Wout,
                  gamma.reshape(1, _D), beta.reshape(1, _D))
    return out


# speedup vs baseline: 3.5369x; 1.7191x over previous
import functools

import jax
import jax.numpy as jnp
from jax import lax
from jax.experimental import pallas as pl
from jax.experimental.pallas import tpu as pltpu
from jax.experimental.pallas import tpu_sc as plsc

_N = 10000
_E = 320000
_D = 128
_NB = 10
_MAXR = 3.5
_NNEI = 32.0
_HID = 100

_NCORE = 2
_NSUB = 16
_NW = _NCORE * _NSUB
_EPW = _E // _NW
_CHI = 80
_IDXROWS = _E // _CHI
_RPW = _IDXROWS // _NW
_GRP = 5
_NGRP = _RPW // _GRP
_GE = _GRP * _CHI
_NPAD = 10240
_NPT = _NPAD // _NSUB

@functools.cache
def _sc_gather_fn():
    mesh = plsc.VectorSubcoreMesh(core_axis_name="c", subcore_axis_name="s",
                                  num_cores=_NCORE, num_subcores=_NSUB)
    return pl.kernel(
        _sc_gather_body,
        out_type=jax.ShapeDtypeStruct((_E, _D), jnp.float32),
        mesh=mesh,
        scratch_types=[
            pltpu.VMEM((_RPW, _CHI), jnp.int32),
            pltpu.VMEM((_GE, _D), jnp.float32),
            pltpu.SemaphoreType.DMA,
        ],
    )


def _sc_gather(x, src):
    return _sc_gather_fn()(x, src)


def _sc_gather_body(x_hbm, src_hbm, xg_hbm, idxv, buf, sem):
    cid = lax.axis_index("c")
    sid = lax.axis_index("s")
    wid = cid * _NSUB + sid
    pltpu.sync_copy(src_hbm.at[wid], idxv)

    def grp(g, carry):
        descs = []
        for k in range(_GRP):
            descs.append(pltpu.async_copy(
                x_hbm.at[idxv.at[g * _GRP + k]],
                buf.at[pl.ds(k * _CHI, _CHI)], sem))
        for d in descs:
            d.wait()
        pltpu.sync_copy(buf, xg_hbm.at[pl.ds(wid * _EPW + g * _GE, _GE)])
        return carry

    lax.fori_loop(0, _NGRP, grp, 0)


@functools.cache
def _sc_scatter_fn():
    mesh = plsc.VectorSubcoreMesh(core_axis_name="c", subcore_axis_name="s",
                                  num_cores=_NCORE, num_subcores=_NSUB)
    return pl.kernel(
        _sc_scatter_body,
        out_type=jax.ShapeDtypeStruct((_NCORE * _NPAD, _D), jnp.float32),
        mesh=mesh,
        scratch_types=[
            pltpu.VMEM((_RPW, _CHI), jnp.int32),
            pltpu.VMEM((_CHI, _D), jnp.float32),
            pltpu.VMEM_SHARED((_NPAD, _D), jnp.float32),
            pltpu.SemaphoreType.DMA,
        ],
    )


def _sc_scatter(m, dst, zer):
    return _sc_scatter_fn()(m, dst, zer)


def _sc_scatter_body(m_hbm, dst_hbm, zer_hbm, out_hbm, idxv, buf, aggsh, sem):
    cid = lax.axis_index("c")
    sid = lax.axis_index("s")
    wid = cid * _NSUB + sid
    pltpu.sync_copy(zer_hbm, aggsh.at[pl.ds(sid * _NPT, _NPT)])
    pltpu.sync_copy(dst_hbm.at[wid], idxv)
    plsc.subcore_barrier()

    def grp(g, carry):
        pltpu.sync_copy(m_hbm.at[pl.ds(wid * _EPW + g * _CHI, _CHI)], buf)
        pltpu.sync_copy(buf, aggsh.at[idxv.at[g]], add=True)
        return carry

    lax.fori_loop(0, _RPW, grp, 0)
    plsc.subcore_barrier()
    for t in range(8):
        off = sid * _NPT + t * _CHI
        pltpu.sync_copy(aggsh.at[pl.ds(off, _CHI)], buf)
        pltpu.sync_copy(buf, out_hbm.at[pl.ds(cid * _NPAD + off, _CHI)])


_BE = 6400


def _edge_body(ev_ref, xg_ref, fcw1_ref, fcb1_ref, fcw2x_ref, fcw2s_ref,
               wsh_ref, m_ref):
    ev = ev_ref[...]
    l2 = jnp.sum(ev * ev, axis=1, keepdims=True)
    ln = jnp.sqrt(l2)
    unit = ev / (ln + 1e-12)
    ea = jnp.concatenate(
        [jnp.ones((_BE, 1), jnp.float32), jnp.sqrt(3.0) * unit], axis=1)
    step = _MAXR / (_NB + 1)
    k = lax.broadcasted_iota(jnp.int32, (_BE, _NB), 1).astype(jnp.float32)
    diff = (ln - (k + 1.0) * step) / step
    mask = ((diff > -1.0) & (diff < 1.0)).astype(jnp.float32)
    u2 = (jnp.pi / 2.0 * diff) ** 2
    cosu = 1.0 + u2 * (-0.5 + u2 * (1.0 / 24.0 + u2 * (-1.0 / 720.0
                                                       + u2 / 40320.0)))
    emb = cosu * mask * (_NB ** 0.5)
    h = jnp.maximum(
        jnp.dot(emb, fcw1_ref[...], preferred_element_type=jnp.float32)
        + fcb1_ref[...], 0.0)
    a = jnp.dot(h, fcw2x_ref[...], preferred_element_type=jnp.float32)
    c = (jnp.dot(ea, wsh_ref[...], preferred_element_type=jnp.float32)
         * jnp.dot(h, fcw2s_ref[...], preferred_element_type=jnp.float32))
    m_ref[...] = xg_ref[...] * a + c


def _edge_messages(edge_vec, xg, fcw1, fcb1, fcw2x, fcw2s, wsh):
    grid = _E // _BE
    full = lambda shape: pl.BlockSpec(shape, lambda i: (0, 0))
    return pl.pallas_call(
        _edge_body,
        grid=(grid,),
        in_specs=[
            pl.BlockSpec((_BE, 3), lambda i: (i, 0)),
            pl.BlockSpec((_BE, _D), lambda i: (i, 0)),
            full((_NB, _HID)),
            full((1, _HID)),
            full((_HID, _D)),
            full((_HID, _D)),
            full((4, _D)),
        ],
        out_specs=pl.BlockSpec((_BE, _D), lambda i: (i, 0)),
        out_shape=jax.ShapeDtypeStruct((_E, _D), jnp.float32),
    )(edge_vec, xg, fcw1, fcb1, fcw2x, fcw2s, wsh)


def _finish_body(x_ref, p_ref, wself_ref, wout_ref, gamma_ref, beta_ref,
                 o_ref):
    agg = (p_ref[0] + p_ref[1]) * (1.0 / jnp.sqrt(_NNEI))
    out = (jnp.dot(x_ref[...], wself_ref[...],
                   preferred_element_type=jnp.float32)
           + jnp.dot(agg, wout_ref[...], preferred_element_type=jnp.float32))
    mean = jnp.mean(out, axis=0, keepdims=True)
    cent = out - mean
    var = jnp.mean(cent * cent, axis=0, keepdims=True)
    o_ref[...] = gamma_ref[...] * cent * lax.rsqrt(var + 1e-5) + beta_ref[...]


def _finish(x, partials, wself, wout, gamma, beta):
    return pl.pallas_call(
        _finish_body,
        out_shape=jax.ShapeDtypeStruct((_N, _D), jnp.float32),
    )(x, partials, wself, wout, gamma, beta)


def kernel(x, pos, edge_index, edge_vec, batch, fcw1, fcb1, fcw2x, fcw2s,
           Wsh, Wself, Wout, gamma, beta):
    src = edge_index[0].reshape(_NW, _RPW, _CHI)
    dst = edge_index[1].reshape(_NW, _RPW, _CHI)
    xg = _sc_gather(x, src)
    m = _edge_messages(edge_vec, xg, fcw1, fcb1.reshape(1, _HID),
                       fcw2x, fcw2s, Wsh)
    zer = jnp.zeros((_NPT, _D), jnp.float32)
    partials = _sc_scatter(m, dst, zer)
    out = _finish(x, partials.reshape(_NCORE, _NPAD, _D)[:, :_N], Wself, Wout,
                  gamma.reshape(1, _D), beta.reshape(1, _D))
    return out


# speedup vs baseline: 4.0638x; 1.1490x over previous
import functools

import jax
import jax.numpy as jnp
from jax import lax
from jax.experimental import pallas as pl
from jax.experimental.pallas import tpu as pltpu
from jax.experimental.pallas import tpu_sc as plsc

_N = 10000
_E = 320000
_D = 128
_NB = 10
_MAXR = 3.5
_NNEI = 32.0
_HID = 100

_NCORE = 2
_NSUB = 16
_NW = _NCORE * _NSUB
_EPW = _E // _NW
_CHI = 80
_IDXROWS = _E // _CHI
_RPW = _IDXROWS // _NW
_GRP = 5
_NGRP = _RPW // _GRP
_GE = _GRP * _CHI
_NPAD = 10240
_NPT = _NPAD // _NSUB

@functools.cache
def _sc_gather_fn():
    mesh = plsc.VectorSubcoreMesh(core_axis_name="c", subcore_axis_name="s",
                                  num_cores=_NCORE, num_subcores=_NSUB)
    return pl.kernel(
        _sc_gather_body,
        out_type=jax.ShapeDtypeStruct((_E, _D), jnp.float32),
        mesh=mesh,
        scratch_types=[
            pltpu.VMEM((_RPW, _CHI), jnp.int32),
            pltpu.VMEM((_GE, _D), jnp.float32),
            pltpu.VMEM((_GE, _D), jnp.float32),
            pltpu.SemaphoreType.DMA,
            pltpu.SemaphoreType.DMA,
        ],
    )


def _sc_gather(x, src):
    return _sc_gather_fn()(x, src)


def _sc_gather_body(x_hbm, src_hbm, xg_hbm, idxv, buf0, buf1, sem0, sem1):
    cid = lax.axis_index("c")
    sid = lax.axis_index("s")
    wid = cid * _NSUB + sid
    pltpu.sync_copy(src_hbm.at[wid], idxv)
    bufs = (buf0, buf1)
    sems = (sem0, sem1)

    def start(g):
        return [pltpu.async_copy(
            x_hbm.at[idxv.at[g * _GRP + k]],
            bufs[g % 2].at[pl.ds(k * _CHI, _CHI)], sems[g % 2])
            for k in range(_GRP)]

    descs = start(0)
    for g in range(_NGRP):
        nxt = start(g + 1) if g + 1 < _NGRP else []
        for d in descs:
            d.wait()
        pltpu.sync_copy(bufs[g % 2],
                        xg_hbm.at[pl.ds(wid * _EPW + g * _GE, _GE)])
        descs = nxt


@functools.cache
def _sc_scatter_fn():
    mesh = plsc.VectorSubcoreMesh(core_axis_name="c", subcore_axis_name="s",
                                  num_cores=_NCORE, num_subcores=_NSUB)
    return pl.kernel(
        _sc_scatter_body,
        out_type=jax.ShapeDtypeStruct((_NCORE * _NPAD, _D), jnp.float32),
        mesh=mesh,
        scratch_types=[
            pltpu.VMEM((_RPW, _CHI), jnp.int32),
            pltpu.VMEM((_CHI, _D), jnp.float32),
            pltpu.VMEM((_CHI, _D), jnp.float32),
            pltpu.VMEM_SHARED((_NPAD, _D), jnp.float32),
            pltpu.SemaphoreType.DMA,
            pltpu.SemaphoreType.DMA,
        ],
    )


def _sc_scatter(m, dst, zer):
    return _sc_scatter_fn()(m, dst, zer)


def _sc_scatter_body(m_hbm, dst_hbm, zer_hbm, out_hbm, idxv, buf0, buf1,
                     aggsh, sem0, sem1):
    cid = lax.axis_index("c")
    sid = lax.axis_index("s")
    wid = cid * _NSUB + sid
    pltpu.sync_copy(zer_hbm, aggsh.at[pl.ds(sid * _NPT, _NPT)])
    pltpu.sync_copy(dst_hbm.at[wid], idxv)
    plsc.subcore_barrier()
    bufs = (buf0, buf1)
    sems = (sem0, sem1)

    def start(g):
        return pltpu.async_copy(
            m_hbm.at[pl.ds(wid * _EPW + g * _CHI, _CHI)],
            bufs[g % 2], sems[g % 2])

    desc = start(0)
    for g in range(_RPW):
        nxt = start(g + 1) if g + 1 < _RPW else None
        desc.wait()
        pltpu.sync_copy(bufs[g % 2], aggsh.at[idxv.at[g]], add=True)
        desc = nxt
    plsc.subcore_barrier()
    for t in range(8):
        off = sid * _NPT + t * _CHI
        pltpu.sync_copy(aggsh.at[pl.ds(off, _CHI)], buf0)
        pltpu.sync_copy(buf0, out_hbm.at[pl.ds(cid * _NPAD + off, _CHI)])


_BE = 6400


def _edge_body(ev_ref, xg_ref, fcw1_ref, fcb1_ref, fcw2x_ref, fcw2s_ref,
               wsh_ref, m_ref):
    ev = ev_ref[...]
    l2 = jnp.sum(ev * ev, axis=1, keepdims=True)
    ln = jnp.sqrt(l2)
    unit = ev / (ln + 1e-12)
    ea = jnp.concatenate(
        [jnp.ones((_BE, 1), jnp.float32), jnp.sqrt(3.0) * unit], axis=1)
    step = _MAXR / (_NB + 1)
    k = lax.broadcasted_iota(jnp.int32, (_BE, _NB), 1).astype(jnp.float32)
    diff = (ln - (k + 1.0) * step) / step
    mask = ((diff > -1.0) & (diff < 1.0)).astype(jnp.float32)
    u2 = (jnp.pi / 2.0 * diff) ** 2
    cosu = 1.0 + u2 * (-0.5 + u2 * (1.0 / 24.0 + u2 * (-1.0 / 720.0
                                                       + u2 / 40320.0)))
    emb = cosu * mask * (_NB ** 0.5)
    h = jnp.maximum(
        jnp.dot(emb, fcw1_ref[...], preferred_element_type=jnp.float32)
        + fcb1_ref[...], 0.0)
    a = jnp.dot(h, fcw2x_ref[...], preferred_element_type=jnp.float32)
    c = (jnp.dot(ea, wsh_ref[...], preferred_element_type=jnp.float32)
         * jnp.dot(h, fcw2s_ref[...], preferred_element_type=jnp.float32))
    m_ref[...] = xg_ref[...] * a + c


def _edge_messages(edge_vec, xg, fcw1, fcb1, fcw2x, fcw2s, wsh):
    grid = _E // _BE
    full = lambda shape: pl.BlockSpec(shape, lambda i: (0, 0))
    return pl.pallas_call(
        _edge_body,
        grid=(grid,),
        in_specs=[
            pl.BlockSpec((_BE, 3), lambda i: (i, 0)),
            pl.BlockSpec((_BE, _D), lambda i: (i, 0)),
            full((_NB, _HID)),
            full((1, _HID)),
            full((_HID, _D)),
            full((_HID, _D)),
            full((4, _D)),
        ],
        out_specs=pl.BlockSpec((_BE, _D), lambda i: (i, 0)),
        out_shape=jax.ShapeDtypeStruct((_E, _D), jnp.float32),
    )(edge_vec, xg, fcw1, fcb1, fcw2x, fcw2s, wsh)


def _finish_body(x_ref, p_ref, wself_ref, wout_ref, gamma_ref, beta_ref,
                 o_ref):
    agg = (p_ref[0] + p_ref[1]) * (1.0 / jnp.sqrt(_NNEI))
    out = (jnp.dot(x_ref[...], wself_ref[...],
                   preferred_element_type=jnp.float32)
           + jnp.dot(agg, wout_ref[...], preferred_element_type=jnp.float32))
    mean = jnp.mean(out, axis=0, keepdims=True)
    cent = out - mean
    var = jnp.mean(cent * cent, axis=0, keepdims=True)
    o_ref[...] = gamma_ref[...] * cent * lax.rsqrt(var + 1e-5) + beta_ref[...]


def _finish(x, partials, wself, wout, gamma, beta):
    return pl.pallas_call(
        _finish_body,
        out_shape=jax.ShapeDtypeStruct((_N, _D), jnp.float32),
    )(x, partials, wself, wout, gamma, beta)


def kernel(x, pos, edge_index, edge_vec, batch, fcw1, fcb1, fcw2x, fcw2s,
           Wsh, Wself, Wout, gamma, beta):
    src = edge_index[0].reshape(_NW, _RPW, _CHI)
    dst = edge_index[1].reshape(_NW, _RPW, _CHI)
    xg = _sc_gather(x, src)
    m = _edge_messages(edge_vec, xg, fcw1, fcb1.reshape(1, _HID),
                       fcw2x, fcw2s, Wsh)
    zer = jnp.zeros((_NPT, _D), jnp.float32)
    partials = _sc_scatter(m, dst, zer)
    out = _finish(x, partials.reshape(_NCORE, _NPAD, _D)[:, :_N], Wself, Wout,
                  gamma.reshape(1, _D), beta.reshape(1, _D))
    return out
